# baseline (speedup 1.0000x reference)
import jax
import jax.numpy as jnp
from jax.experimental import pallas as pl
from jax.experimental.pallas import tpu as pltpu

N = 2048
K = 5
BR = 1024
BC = 1024
NI = N // BR
NJ = N // BC
_PADR = 8


def _knn_std_kernel(x_ref, y_ref, out_ref, run_ref, acc_ref):
    i = pl.program_id(0)
    j = pl.program_id(1)

    @pl.when(j == 0)
    def _():
        run_ref[...] = jnp.full((_PADR, BR), jnp.inf, jnp.float32)

    x = x_ref[...]
    y = y_ref[...]

    g = jax.lax.dot_general(
        y, x, (((1,), (1,)), ((), ())), preferred_element_type=jnp.float32
    )
    y2 = jnp.sum(y * y, axis=1, keepdims=True)
    st = y2 - 2.0 * g

    cand = jnp.concatenate([run_ref[...], st], axis=0)
    H = _PADR + BC
    iota = jax.lax.broadcasted_iota(jnp.int32, (H, BR), 0)
    row = jax.lax.broadcasted_iota(jnp.int32, (_PADR, BR), 0)
    new_run = jnp.full((_PADR, BR), jnp.inf, jnp.float32)
    for t in range(K):
        m = jnp.min(cand, axis=0, keepdims=True)
        r0 = jnp.min(jnp.where(cand == m, iota, H), axis=0, keepdims=True)
        cand = jnp.where(iota == r0, jnp.inf, cand)
        new_run = jnp.where(row == t, m, new_run)
    run_ref[...] = new_run

    @pl.when(j == NJ - 1)
    def _():
        ones = jnp.ones((8, N), jnp.float32)
        x2 = jax.lax.dot_general(
            ones, x * x, (((1,), (1,)), ((), ())),
            preferred_element_type=jnp.float32,
        )[0:1, :]
        d2 = jnp.maximum(x2 + new_run, 0.0)
        knn = jnp.where(d2 > 0.0, jnp.sqrt(jnp.where(d2 > 0.0, d2, 1.0)), 0.0)
        valid = row < K
        knn = jnp.where(valid, knn, 0.0)
        nb = jnp.float32(BR * K)
        mean_b = jnp.sum(knn) / nb
        dev = jnp.where(valid, knn - mean_b, 0.0)
        m2_b = jnp.sum(dev * dev)

        @pl.when(i == 0)
        def _():
            acc_ref[0] = nb
            acc_ref[1] = mean_b
            acc_ref[2] = m2_b

        @pl.when(i > 0)
        def _():
            na = acc_ref[0]
            mean_a = acc_ref[1]
            m2_a = acc_ref[2]
            n = na + nb
            delta = mean_b - mean_a
            acc_ref[0] = n
            acc_ref[1] = mean_a + delta * (nb / n)
            acc_ref[2] = m2_a + m2_b + delta * delta * (na * nb / n)

        @pl.when(i == NI - 1)
        def _():
            n = acc_ref[0]
            out_ref[...] = jnp.full(
                (1, 1), -jnp.sqrt(acc_ref[2] / (n - 1.0)), jnp.float32
            )


def kernel(latent):
    sample = latent[0]
    out = pl.pallas_call(
        _knn_std_kernel,
        grid=(NI, NJ),
        in_specs=[
            pl.BlockSpec((BR, N), lambda i, j: (i, 0)),
            pl.BlockSpec((BC, N), lambda i, j: (j, 0)),
        ],
        out_specs=pl.BlockSpec((1, 1), lambda i, j: (0, 0)),
        out_shape=jax.ShapeDtypeStruct((1, 1), jnp.float32),
        scratch_shapes=[
            pltpu.VMEM((_PADR, BR), jnp.float32),
            pltpu.SMEM((4,), jnp.float32),
        ],
    )(sample, sample)
    return out[0, 0]


# speedup vs baseline: 1.1265x; 1.1265x over previous
import jax
import jax.numpy as jnp
from jax.experimental import pallas as pl
from jax.experimental.pallas import tpu as pltpu

N = 2048
K = 5
BR = 256
NI = N // BR
_PADR = 8


def _knn_std_kernel(x_ref, y_ref, out_ref, y2_ref, acc_ref):
    i = pl.program_id(0)

    x = x_ref[...]
    y = y_ref[...]

    g = jax.lax.dot_general(
        y, x, (((1,), (1,)), ((), ())), preferred_element_type=jnp.float32
    )

    @pl.when(i == 0)
    def _():
        y2_ref[...] = jnp.sum(y * y, axis=1, keepdims=True)

    y2 = y2_ref[...]
    st = y2 - 2.0 * g

    inf = jnp.full((_PADR, BR), jnp.inf, jnp.float32)
    s = [inf] * K
    for r in range(N // _PADR):
        v = st[r * _PADR:(r + 1) * _PADR, :]
        for t in range(K - 1, -1, -1):
            lo = jnp.minimum(s[t], v)
            v = jnp.maximum(s[t], v)
            s[t] = lo

    cand = jnp.concatenate(s, axis=0)
    H = K * _PADR
    iota = jax.lax.broadcasted_iota(jnp.int32, (H, BR), 0)
    row = jax.lax.broadcasted_iota(jnp.int32, (_PADR, BR), 0)
    sel = jnp.full((_PADR, BR), jnp.inf, jnp.float32)
    for t in range(K):
        m = jnp.min(cand, axis=0, keepdims=True)
        r0 = jnp.min(jnp.where(cand == m, iota, H), axis=0, keepdims=True)
        cand = jnp.where(iota == r0, jnp.inf, cand)
        sel = jnp.where(row == t, m, sel)

    ones = jnp.ones((8, N), jnp.float32)
    x2 = jax.lax.dot_general(
        ones, x * x, (((1,), (1,)), ((), ())),
        preferred_element_type=jnp.float32,
    )[0:1, :]
    d2 = jnp.maximum(x2 + sel, 0.0)
    knn = jnp.where(d2 > 0.0, jnp.sqrt(jnp.where(d2 > 0.0, d2, 1.0)), 0.0)
    valid = row < K
    knn = jnp.where(valid, knn, 0.0)
    nb = jnp.float32(BR * K)
    mean_b = jnp.sum(knn) / nb
    dev = jnp.where(valid, knn - mean_b, 0.0)
    m2_b = jnp.sum(dev * dev)

    @pl.when(i == 0)
    def _():
        acc_ref[0] = nb
        acc_ref[1] = mean_b
        acc_ref[2] = m2_b

    @pl.when(i > 0)
    def _():
        na = acc_ref[0]
        mean_a = acc_ref[1]
        m2_a = acc_ref[2]
        n = na + nb
        delta = mean_b - mean_a
        acc_ref[0] = n
        acc_ref[1] = mean_a + delta * (nb / n)
        acc_ref[2] = m2_a + m2_b + delta * delta * (na * nb / n)

    @pl.when(i == NI - 1)
    def _():
        n = acc_ref[0]
        out_ref[...] = jnp.full(
            (1, 1), -jnp.sqrt(acc_ref[2] / (n - 1.0)), jnp.float32
        )


def kernel(latent):
    sample = latent[0]
    out = pl.pallas_call(
        _knn_std_kernel,
        grid=(NI,),
        in_specs=[
            pl.BlockSpec((BR, N), lambda i: (i, 0)),
            pl.BlockSpec((N, N), lambda i: (0, 0)),
        ],
        out_specs=pl.BlockSpec((1, 1), lambda i: (0, 0)),
        out_shape=jax.ShapeDtypeStruct((1, 1), jnp.float32),
        scratch_shapes=[
            pltpu.VMEM((N, 1), jnp.float32),
            pltpu.SMEM((4,), jnp.float32),
        ],
    )(sample, sample)
    return out[0, 0]
